# Initial kernel scaffold; baseline (speedup 1.0000x reference)
import jax
import jax.numpy as jnp
from jax.experimental import pallas as pl


def kernel(x, edge_index, W1, b1, W2, b2, Wr, br):
    raise NotImplementedError("write your pallas kernel here")


# speedup vs baseline: 13.7121x; 13.7121x over previous
import functools

import jax
import jax.numpy as jnp
from jax import lax
from jax.experimental import pallas as pl
from jax.experimental.pallas import tpu as pltpu
from jax.experimental.pallas import tpu_sc as plsc

N = 10000
D = 256
DH = 128
NC = 2
NS = 16
B = 128
H = 10240
RPT = 632
RPT_LAST = N - (NS - 1) * RPT
ZPT = H // NS

@functools.cache
def _mesh():
    return plsc.VectorSubcoreMesh(core_axis_name="c", subcore_axis_name="s",
                                  num_cores=NC, num_subcores=NS)


def _deg_body(dst_t, zeros_h, ones_h, out, hist, dst_v, ones_v):
    c = lax.axis_index("c")
    s = lax.axis_index("s")
    nb = dst_t.shape[1]
    half = nb // 2
    pltpu.sync_copy(zeros_h.at[pl.ds(s * ZPT, ZPT)], hist.at[pl.ds(s * ZPT, ZPT)])
    pltpu.sync_copy(dst_t.at[s], dst_v)
    pltpu.sync_copy(ones_h, ones_v)
    plsc.subcore_barrier()
    lo = c * half
    hi = jnp.where(c == 0, half, nb)

    def body(j, carry):
        pltpu.sync_copy(ones_v, hist.at[dst_v.at[j]], add=True)
        return carry

    lax.fori_loop(lo, hi, body, 0)
    plsc.subcore_barrier()
    pltpu.sync_copy(hist.at[pl.ds(s * ZPT, ZPT)],
                    out.at[pl.ds(c * H + s * ZPT, ZPT)])


def _deg_call(dst_t, zeros_h, ones_h):
    nb = dst_t.shape[1]
    return pl.kernel(
        _deg_body,
        out_type=jax.ShapeDtypeStruct((NC * H,), jnp.float32),
        mesh=_mesh(),
        scratch_types=[
            pltpu.VMEM_SHARED((H,), jnp.float32),
            pltpu.VMEM((nb, B), jnp.int32),
            pltpu.VMEM((B,), jnp.float32),
        ],
    )(dst_t, zeros_h, ones_h)


def _agg_body(gstack, src_w, dst_t, out, agg, src_v, dst_v, rows_v, sem):
    c = lax.axis_index("c")
    s = lax.axis_index("s")
    w = c * NS + s
    nb = dst_t.shape[1]
    @pl.when(s < NS - 1)
    def _():
        pltpu.sync_copy(gstack.at[pl.ds(c * N + s * RPT, RPT)],
                        agg.at[pl.ds(s * RPT, RPT)])

    @pl.when(s == NS - 1)
    def _():
        pltpu.sync_copy(gstack.at[pl.ds(c * N + (NS - 1) * RPT, RPT_LAST)],
                        agg.at[pl.ds((NS - 1) * RPT, RPT_LAST)])
    pltpu.sync_copy(src_w.at[w], src_v)
    pltpu.sync_copy(dst_t.at[s], dst_v)
    plsc.subcore_barrier()

    def body(j, carry):
        pltpu.async_copy(gstack.at[src_v.at[j]], rows_v, sem).wait()
        pltpu.sync_copy(rows_v, agg.at[dst_v.at[j]], add=True)
        return carry

    lax.fori_loop(0, nb, body, 0)
    plsc.subcore_barrier()

    @pl.when(s < NS - 1)
    def _():
        pltpu.sync_copy(agg.at[pl.ds(s * RPT, RPT)],
                        out.at[pl.ds(c * N + s * RPT, RPT)])

    @pl.when(s == NS - 1)
    def _():
        pltpu.sync_copy(agg.at[pl.ds((NS - 1) * RPT, RPT_LAST)],
                        out.at[pl.ds(c * N + (NS - 1) * RPT, RPT_LAST)])


def _agg_call(gstack, src_w, dst_t):
    nb = dst_t.shape[1]
    return pl.kernel(
        _agg_body,
        out_type=jax.ShapeDtypeStruct((NC * N, DH), jnp.float32),
        mesh=_mesh(),
        scratch_types=[
            pltpu.VMEM_SHARED((H, DH), jnp.float32),
            pltpu.VMEM((nb, B), jnp.int32),
            pltpu.VMEM((nb, B), jnp.int32),
            pltpu.VMEM((B, DH), jnp.float32),
            pltpu.SemaphoreType.DMA,
        ],
    )(gstack, src_w, dst_t)


_PREC = lax.Precision.HIGHEST


def _tc1_body(x_b, w1, hist_b, g_out, dinv_out):
    deg = 1.0 + jnp.sum(hist_b[...], axis=1, keepdims=True)
    dinv = lax.rsqrt(deg)
    h = jnp.dot(x_b[...], w1[...], preferred_element_type=jnp.float32,
                precision=_PREC)
    g = h * dinv
    g_out[0, :, :] = g[:, :DH]
    g_out[1, :, :] = g[:, DH:]
    dinv_out[...] = dinv


def _tc1_call(x, w1, hist_t, bm=2000):
    grid = (N // bm,)
    return pl.pallas_call(
        _tc1_body,
        grid=grid,
        in_specs=[
            pl.BlockSpec((bm, D), lambda i: (i, 0)),
            pl.BlockSpec((D, D), lambda i: (0, 0)),
            pl.BlockSpec((bm, NC), lambda i: (i, 0)),
        ],
        out_specs=[
            pl.BlockSpec((NC, bm, DH), lambda i: (0, i, 0)),
            pl.BlockSpec((bm, 1), lambda i: (i, 0)),
        ],
        out_shape=[
            jax.ShapeDtypeStruct((NC, N, DH), jnp.float32),
            jax.ShapeDtypeStruct((N, 1), jnp.float32),
        ],
    )(x, w1, hist_t)


def _leaky(v):
    return jnp.where(v >= 0, v, 0.01 * v)


def _tc2_body(agg_b, dinv_b, b1, w2, g_out):
    a = jnp.concatenate([agg_b[0, :, :], agg_b[1, :, :]], axis=1)
    dinv = dinv_b[...]
    pre = a * dinv + b1[...]
    hact = _leaky(pre)
    h2 = jnp.dot(hact, w2[...], preferred_element_type=jnp.float32,
                 precision=_PREC)
    g = h2 * dinv
    g_out[0, :, :] = g[:, :DH]
    g_out[1, :, :] = g[:, DH:]


def _tc2_call(agg, dinv, b1, w2, bm=2000):
    grid = (N // bm,)
    return pl.pallas_call(
        _tc2_body,
        grid=grid,
        in_specs=[
            pl.BlockSpec((NC, bm, DH), lambda i: (0, i, 0)),
            pl.BlockSpec((bm, 1), lambda i: (i, 0)),
            pl.BlockSpec((1, D), lambda i: (0, 0)),
            pl.BlockSpec((D, D), lambda i: (0, 0)),
        ],
        out_specs=pl.BlockSpec((NC, bm, DH), lambda i: (0, i, 0)),
        out_shape=jax.ShapeDtypeStruct((NC, N, DH), jnp.float32),
    )(agg, dinv, b1, w2)


def _tc3_body(agg_b, dinv_b, b2, wr_row, br, y_out):
    a = jnp.concatenate([agg_b[0, :, :], agg_b[1, :, :]], axis=1)
    pre = a * dinv_b[...] + b2[...]
    hact = _leaky(pre)
    z = jnp.sum(hact * wr_row[...], axis=1, keepdims=True) + br[...]
    y_out[...] = 1.0 / (1.0 + jnp.exp(-z))


def _tc3_call(agg, dinv, b2, wr_row, br, bm=2000):
    grid = (N // bm,)
    return pl.pallas_call(
        _tc3_body,
        grid=grid,
        in_specs=[
            pl.BlockSpec((NC, bm, DH), lambda i: (0, i, 0)),
            pl.BlockSpec((bm, 1), lambda i: (i, 0)),
            pl.BlockSpec((1, D), lambda i: (0, 0)),
            pl.BlockSpec((1, D), lambda i: (0, 0)),
            pl.BlockSpec((1, 1), lambda i: (0, 0)),
        ],
        out_specs=pl.BlockSpec((bm, 1), lambda i: (i, 0)),
        out_shape=jax.ShapeDtypeStruct((N, 1), jnp.float32),
    )(agg, dinv, b2, wr_row, br)


def kernel(x, edge_index, W1, b1, W2, b2, Wr, br):
    ei = edge_index.astype(jnp.int32)
    src, dst = ei[0], ei[1]
    e = src.shape[0]
    nb = -(-e // (NS * B))
    e_pad = NS * B * nb
    pad = e_pad - e
    pidx = jnp.arange(pad, dtype=jnp.int32)
    src_p = jnp.concatenate([src, pidx % N])
    dst_p = jnp.concatenate([dst, N + pidx % (H - N)])
    src_w = jnp.stack([src_p, src_p + N]).reshape(NC * NS, nb, B)
    dst_t = dst_p.reshape(NS, nb, B)

    zeros_h = jnp.zeros((H,), jnp.float32)
    ones_h = jnp.ones((B,), jnp.float32)
    partials = _deg_call(dst_t, zeros_h, ones_h)
    hist_t = partials.reshape(NC, H)[:, :N].T

    g1, dinv = _tc1_call(x, W1, hist_t)
    agg1 = _agg_call(g1.reshape(NC * N, DH), src_w, dst_t)
    g2 = _tc2_call(agg1.reshape(NC, N, DH), dinv, b1.reshape(1, D), W2)
    agg2 = _agg_call(g2.reshape(NC * N, DH), src_w, dst_t)
    y = _tc3_call(agg2.reshape(NC, N, DH), dinv, b2.reshape(1, D),
                  Wr.reshape(1, D), br.reshape(1, 1))
    return y
